# baseline (speedup 1.0000x reference)
import functools

import jax
import jax.numpy as jnp
from jax import lax
from jax.experimental import pallas as pl
from jax.experimental.pallas import tpu as pltpu
from jax.experimental.pallas import tpu_sc as plsc

N_NODES = 10000
N_EDGES = 320000
D = 128
D_EDGE = 16

NC = 2
NS = 16
NW = NC * NS
E_PER_W = N_EDGES // NW
CHUNK = 80
N_CHUNKS = E_PER_W // CHUNK
N_PAIR = N_CHUNKS // 2

AGG_BE = 1280
AGG_NB = N_EDGES // AGG_BE
AGG_W = 64


def _proj_body(n_ref, ws_ref, wr_ref, ps_ref, pr_ref):
    x = n_ref[...]
    ps_ref[...] = jnp.dot(x, ws_ref[...], preferred_element_type=jnp.float32)
    pr_ref[...] = jnp.dot(x, wr_ref[...], preferred_element_type=jnp.float32)


def _node_body(n_ref, a_ref, w1_ref, w2_ref, b_ref, o_ref):
    o_ref[...] = jnp.maximum(
        jnp.dot(n_ref[...], w1_ref[...], preferred_element_type=jnp.float32)
        + jnp.dot(a_ref[...], w2_ref[...], preferred_element_type=jnp.float32)
        + b_ref[...],
        0.0,
    )


def _sc_body(ps_hbm, s_hbm,
             gp_hbm,
             sidx_f,
             b0, b1, b2, b3,
             sg0, sg1, sg2, sg3,
             sw0, sw1, sw2, sw3):
    c = lax.axis_index("c")
    s = lax.axis_index("s")
    wid = s * NC + c
    edge0 = wid * E_PER_W
    bufs = (b0, b1, b2, b3)
    gsems = (sg0, sg1, sg2, sg3)
    wsems = (sw0, sw1, sw2, sw3)

    def mk_g(k, j):
        return pltpu.make_async_copy(
            ps_hbm.at[sidx_f.at[pl.ds(k * CHUNK, CHUNK)]], bufs[j], gsems[j])

    def mk_w(k, j):
        base = edge0 + k * CHUNK
        return pltpu.make_async_copy(
            bufs[j], gp_hbm.at[pl.ds(base, CHUNK)], wsems[j])

    pltpu.sync_copy(s_hbm.at[pl.ds(edge0, E_PER_W)], sidx_f)

    for j in range(4):
        mk_g(j, j).start()

    last = N_CHUNKS - 1

    def body(i, _):
        for j in range(4):
            k = 4 * i + j
            mk_g(k, j).wait()
            mk_w(k, j).start()
            k2 = k - 2
            j2 = (j + 2) % 4

            @pl.when(k2 >= 0)
            def _():
                mk_w(k2, j2).wait()

            @pl.when((k2 >= 0) & (k2 + 4 <= last))
            def _():
                mk_g(k2 + 4, j2).start()
        return 0
    lax.fori_loop(0, (N_CHUNKS - 1) // 4, body, 0)

    mk_w(last - 2, 2).wait()
    mk_w(last - 1, 3).wait()
    mk_g(last, 0).wait()
    mk_w(last, 0).start()
    mk_w(last, 0).wait()


def _edge_agg_body(e_ref, w_ref, b_ref, gp_ref, r_ref, pr_ref,
                   ne_ref, agg_ref):
    i = pl.program_id(0)

    @pl.when(i == 0)
    def _():
        agg_ref[...] = jnp.zeros_like(agg_ref)

    pre = (
        jax.lax.dot_general(e_ref[...], w_ref[...], (((0,), (0,)), ((), ())),
                            preferred_element_type=jnp.float32)
        + b_ref[...] + gp_ref[...]
    )
    r = r_ref[0, 0, :]
    rr = r[:, None]
    cols = jax.lax.broadcasted_iota(jnp.int32, (AGG_BE, AGG_W), 1)
    r0 = r[0]
    rmax = r[AGG_BE - 1]
    w0 = jnp.minimum((r0 // 8) * 8, N_NODES - AGG_W)
    nwin = (rmax - w0) // AGG_W + 1

    def mk_oh(k):
        lob = w0 + k * AGG_W
        wk = jnp.minimum(lob, N_NODES - AGG_W)
        oh = ((rr - wk == cols) & (rr >= lob)).astype(jnp.bfloat16)
        return wk, oh

    def expand(k, acc):
        wk, oh = mk_oh(k)
        prw = pr_ref[pl.ds(wk, AGG_W), :].astype(jnp.bfloat16)
        return acc + jax.lax.dot_general(
            oh, prw, (((1,), (0,)), ((), ())),
            preferred_element_type=jnp.float32)
    ne = jnp.maximum(lax.fori_loop(0, nwin, expand, pre), 0.0)
    ne_ref[...] = ne
    neb = ne.astype(jnp.bfloat16)

    def agg_win(k, _):
        wk, oh = mk_oh(k)
        part = jax.lax.dot_general(
            oh, neb, (((0,), (0,)), ((), ())),
            preferred_element_type=jnp.float32)
        agg_ref[pl.ds(wk, AGG_W), :] += part
        return 0
    lax.fori_loop(0, nwin, agg_win, 0)


@functools.lru_cache(maxsize=None)
def _get_sc_edges():
  return pl.kernel(
    _sc_body,
    out_type=jax.ShapeDtypeStruct((N_EDGES, D), jnp.float32),
    mesh=plsc.VectorSubcoreMesh(core_axis_name="c", subcore_axis_name="s",
                                num_cores=NC, num_subcores=NS),
    compiler_params=pltpu.CompilerParams(use_tc_tiling_on_sc=True),
    scratch_types=(
        [pltpu.VMEM((E_PER_W,), jnp.int32)]
        + [pltpu.VMEM((CHUNK, D), jnp.float32)] * 4
        + [pltpu.SemaphoreType.DMA] * 8
    ),
  )


@jax.jit
def _run(nodes, edges, senders, receivers, W_edge, b_edge, W_node, b_node):
    w1 = W_edge[:D_EDGE]
    w_es = W_edge[D_EDGE:D_EDGE + D]
    w_er = W_edge[D_EDGE + D:]
    wn1 = W_node[:D]
    wn2 = W_node[D:]
    be = b_edge.reshape(1, D)
    bn = b_node.reshape(1, D)

    nb = 1000
    ps, pr = pl.pallas_call(
        _proj_body,
        grid=(N_NODES // nb,),
        in_specs=[
            pl.BlockSpec((nb, D), lambda i: (i, 0)),
            pl.BlockSpec((D, D), lambda i: (0, 0)),
            pl.BlockSpec((D, D), lambda i: (0, 0)),
        ],
        out_specs=[
            pl.BlockSpec((nb, D), lambda i: (i, 0)),
            pl.BlockSpec((nb, D), lambda i: (i, 0)),
        ],
        out_shape=[
            jax.ShapeDtypeStruct((N_NODES, D), jnp.float32),
            jax.ShapeDtypeStruct((N_NODES, D), jnp.float32),
        ],
    )(nodes, w_es, w_er)

    gps = _get_sc_edges()(ps, senders)

    r3 = receivers.reshape(AGG_NB, 1, AGG_BE)
    new_edges, agg = pl.pallas_call(
        _edge_agg_body,
        grid=(AGG_NB,),
        in_specs=[
            pl.BlockSpec((D_EDGE, AGG_BE), lambda i: (0, i)),
            pl.BlockSpec((D_EDGE, D), lambda i: (0, 0)),
            pl.BlockSpec((1, D), lambda i: (0, 0)),
            pl.BlockSpec((AGG_BE, D), lambda i: (i, 0)),
            pl.BlockSpec((1, 1, AGG_BE), lambda i: (i, 0, 0)),
            pl.BlockSpec((N_NODES, D), lambda i: (0, 0)),
        ],
        out_specs=[
            pl.BlockSpec((AGG_BE, D), lambda i: (i, 0)),
            pl.BlockSpec((N_NODES, D), lambda i: (0, 0)),
        ],
        out_shape=[
            jax.ShapeDtypeStruct((N_EDGES, D), jnp.float32),
            jax.ShapeDtypeStruct((N_NODES, D), jnp.float32),
        ],
        compiler_params=pltpu.CompilerParams(
            fuse_transposed_lhs_in_matmul=True),
    )(jnp.swapaxes(edges, 0, 1), w1, be, gps, r3, pr)

    new_nodes = pl.pallas_call(
        _node_body,
        grid=(N_NODES // nb,),
        in_specs=[
            pl.BlockSpec((nb, D), lambda i: (i, 0)),
            pl.BlockSpec((nb, D), lambda i: (i, 0)),
            pl.BlockSpec((D, D), lambda i: (0, 0)),
            pl.BlockSpec((D, D), lambda i: (0, 0)),
            pl.BlockSpec((1, D), lambda i: (0, 0)),
        ],
        out_specs=pl.BlockSpec((nb, D), lambda i: (i, 0)),
        out_shape=jax.ShapeDtypeStruct((N_NODES, D), jnp.float32),
    )(nodes, agg, wn1, wn2, bn)

    return new_nodes, new_edges


def kernel(nodes, edges, senders, receivers, W_edge, b_edge, W_node, b_node):
    return _run(nodes, edges, senders, receivers,
                W_edge, b_edge, W_node, b_node)


# speedup vs baseline: 1.0375x; 1.0375x over previous
import functools

import jax
import jax.numpy as jnp
from jax import lax
from jax.experimental import pallas as pl
from jax.experimental.pallas import tpu as pltpu
from jax.experimental.pallas import tpu_sc as plsc

N_NODES = 10000
N_EDGES = 320000
D = 128
D_EDGE = 16

NC = 2
NS = 16
NW = NC * NS
E_HALF = N_EDGES // 2
E_PER_W = E_HALF // NW
CHUNK = 40
N_CHUNKS = E_PER_W // CHUNK

AGG_BE = 1280
AGG_NB = N_EDGES // AGG_BE
AGG_W = 64


def _proj_body(n_ref, ws_ref, wr_ref, ps_ref, pr_ref):
    x = n_ref[...]
    ps_ref[...] = jnp.dot(x, ws_ref[...], preferred_element_type=jnp.float32)
    pr_ref[...] = jnp.dot(x, wr_ref[...], preferred_element_type=jnp.float32)


def _node_body(n_ref, a_ref, w1_ref, w2_ref, b_ref, o_ref):
    o_ref[...] = jnp.maximum(
        jnp.dot(n_ref[...], w1_ref[...], preferred_element_type=jnp.float32)
        + jnp.dot(a_ref[...], w2_ref[...], preferred_element_type=jnp.float32)
        + b_ref[...],
        0.0,
    )


def _sc_body(ps_hbm, s_hbm,
             gp_hbm,
             sidx_f,
             b0, b1, b2, b3,
             sg0, sg1, sg2, sg3,
             sw0, sw1, sw2, sw3):
    c = lax.axis_index("c")
    s = lax.axis_index("s")
    wid = s * NC + c
    edge0 = wid * E_PER_W
    bufs = (b0, b1, b2, b3)
    gsems = (sg0, sg1, sg2, sg3)
    wsems = (sw0, sw1, sw2, sw3)

    def mk_g(k, j):
        return pltpu.make_async_copy(
            ps_hbm.at[sidx_f.at[pl.ds(k * CHUNK, CHUNK)]], bufs[j], gsems[j])

    def mk_w(k, j):
        base = edge0 + k * CHUNK
        return pltpu.make_async_copy(
            bufs[j], gp_hbm.at[pl.ds(base, CHUNK)], wsems[j])

    pltpu.sync_copy(s_hbm.at[pl.ds(edge0, E_PER_W)], sidx_f)

    for j in range(4):
        mk_g(j, j).start()

    last = N_CHUNKS - 1

    def body(i, _):
        for j in range(4):
            k = 4 * i + j
            mk_g(k, j).wait()
            mk_w(k, j).start()
            k2 = k - 2
            j2 = (j + 2) % 4

            @pl.when(k2 >= 0)
            def _():
                mk_w(k2, j2).wait()

            @pl.when((k2 >= 0) & (k2 + 4 <= last))
            def _():
                mk_g(k2 + 4, j2).start()
        return 0
    lax.fori_loop(0, (N_CHUNKS - 1) // 4, body, 0)

    mk_w(last - 2, 2).wait()
    mk_w(last - 1, 3).wait()
    mk_g(last, 0).wait()
    mk_w(last, 0).start()
    mk_w(last, 0).wait()


def _edge_agg_body(first, *refs):
    if first:
        (e_ref, w_ref, b_ref, gp_ref, r_ref, pr_ref,
         ne_ref, agg_ref) = refs
    else:
        (e_ref, w_ref, b_ref, gp_ref, r_ref, pr_ref, _nein_ref, aggin_ref,
         ne_ref, agg_ref) = refs
    i = pl.program_id(0)

    @pl.when(i == 0)
    def _():
        if first:
            agg_ref[...] = jnp.zeros_like(agg_ref)
        else:
            agg_ref[...] = aggin_ref[...]

    pre = (
        jax.lax.dot_general(e_ref[...], w_ref[...], (((0,), (0,)), ((), ())),
                            preferred_element_type=jnp.float32)
        + b_ref[...] + gp_ref[...]
    )
    r = r_ref[0, 0, :]
    rr = r[:, None]
    cols = jax.lax.broadcasted_iota(jnp.int32, (AGG_BE, AGG_W), 1)
    r0 = r[0]
    rmax = r[AGG_BE - 1]
    w0 = jnp.minimum((r0 // 8) * 8, N_NODES - AGG_W)
    nwin = (rmax - w0) // AGG_W + 1

    def mk_oh(k):
        lob = w0 + k * AGG_W
        wk = jnp.minimum(lob, N_NODES - AGG_W)
        oh = ((rr - wk == cols) & (rr >= lob)).astype(jnp.bfloat16)
        return wk, oh

    def expand(k, acc):
        wk, oh = mk_oh(k)
        prw = pr_ref[pl.ds(wk, AGG_W), :].astype(jnp.bfloat16)
        return acc + jax.lax.dot_general(
            oh, prw, (((1,), (0,)), ((), ())),
            preferred_element_type=jnp.float32)
    ne = jnp.maximum(lax.fori_loop(0, nwin, expand, pre), 0.0)
    ne_ref[...] = ne
    neb = ne.astype(jnp.bfloat16)

    def agg_win(k, _):
        wk, oh = mk_oh(k)
        part = jax.lax.dot_general(
            oh, neb, (((0,), (0,)), ((), ())),
            preferred_element_type=jnp.float32)
        agg_ref[pl.ds(wk, AGG_W), :] += part
        return 0
    lax.fori_loop(0, nwin, agg_win, 0)


@functools.lru_cache(maxsize=None)
def _get_sc_edges():
  return pl.kernel(
    _sc_body,
    out_type=jax.ShapeDtypeStruct((E_HALF, D), jnp.float32),
    mesh=plsc.VectorSubcoreMesh(core_axis_name="c", subcore_axis_name="s",
                                num_cores=NC, num_subcores=NS),
    compiler_params=pltpu.CompilerParams(use_tc_tiling_on_sc=True),
    scratch_types=(
        [pltpu.VMEM((E_PER_W,), jnp.int32)]
        + [pltpu.VMEM((CHUNK, D), jnp.float32)] * 4
        + [pltpu.SemaphoreType.DMA] * 8
    ),
  )


@jax.jit
def _run(nodes, edges, senders, receivers, W_edge, b_edge, W_node, b_node):
    w1 = W_edge[:D_EDGE]
    w_es = W_edge[D_EDGE:D_EDGE + D]
    w_er = W_edge[D_EDGE + D:]
    wn1 = W_node[:D]
    wn2 = W_node[D:]
    be = b_edge.reshape(1, D)
    bn = b_node.reshape(1, D)

    nb = 1000
    ps, pr = pl.pallas_call(
        _proj_body,
        grid=(N_NODES // nb,),
        in_specs=[
            pl.BlockSpec((nb, D), lambda i: (i, 0)),
            pl.BlockSpec((D, D), lambda i: (0, 0)),
            pl.BlockSpec((D, D), lambda i: (0, 0)),
        ],
        out_specs=[
            pl.BlockSpec((nb, D), lambda i: (i, 0)),
            pl.BlockSpec((nb, D), lambda i: (i, 0)),
        ],
        out_shape=[
            jax.ShapeDtypeStruct((N_NODES, D), jnp.float32),
            jax.ShapeDtypeStruct((N_NODES, D), jnp.float32),
        ],
    )(nodes, w_es, w_er)

    gps0 = _get_sc_edges()(ps, senders[:E_HALF])
    gps1 = _get_sc_edges()(ps, senders[E_HALF:])

    eT = jnp.swapaxes(edges, 0, 1)
    r3 = receivers.reshape(AGG_NB, 1, AGG_BE)
    nbh = AGG_NB // 2
    out_shapes = [
        jax.ShapeDtypeStruct((N_EDGES, D), jnp.float32),
        jax.ShapeDtypeStruct((N_NODES, D), jnp.float32),
    ]
    ne_a, agg_a = pl.pallas_call(
        functools.partial(_edge_agg_body, True),
        grid=(nbh,),
        in_specs=[
            pl.BlockSpec((D_EDGE, AGG_BE), lambda i: (0, i)),
            pl.BlockSpec((D_EDGE, D), lambda i: (0, 0)),
            pl.BlockSpec((1, D), lambda i: (0, 0)),
            pl.BlockSpec((AGG_BE, D), lambda i: (i, 0)),
            pl.BlockSpec((1, 1, AGG_BE), lambda i: (i, 0, 0)),
            pl.BlockSpec((N_NODES, D), lambda i: (0, 0)),
        ],
        out_specs=[
            pl.BlockSpec((AGG_BE, D), lambda i: (i, 0)),
            pl.BlockSpec((N_NODES, D), lambda i: (0, 0)),
        ],
        out_shape=out_shapes,
    )(eT, w1, be, gps0, r3, pr)
    new_edges, agg = pl.pallas_call(
        functools.partial(_edge_agg_body, False),
        grid=(nbh,),
        in_specs=[
            pl.BlockSpec((D_EDGE, AGG_BE), lambda i: (0, i + nbh)),
            pl.BlockSpec((D_EDGE, D), lambda i: (0, 0)),
            pl.BlockSpec((1, D), lambda i: (0, 0)),
            pl.BlockSpec((AGG_BE, D), lambda i: (i, 0)),
            pl.BlockSpec((1, 1, AGG_BE), lambda i: (i + nbh, 0, 0)),
            pl.BlockSpec((N_NODES, D), lambda i: (0, 0)),
            pl.BlockSpec((8, D), lambda i: (0, 0)),
            pl.BlockSpec((N_NODES, D), lambda i: (0, 0)),
        ],
        out_specs=[
            pl.BlockSpec((AGG_BE, D), lambda i: (i + nbh, 0)),
            pl.BlockSpec((N_NODES, D), lambda i: (0, 0)),
        ],
        out_shape=out_shapes,
        input_output_aliases={6: 0},
    )(eT, w1, be, gps1, r3, pr, ne_a, agg_a)

    new_nodes = pl.pallas_call(
        _node_body,
        grid=(N_NODES // nb,),
        in_specs=[
            pl.BlockSpec((nb, D), lambda i: (i, 0)),
            pl.BlockSpec((nb, D), lambda i: (i, 0)),
            pl.BlockSpec((D, D), lambda i: (0, 0)),
            pl.BlockSpec((D, D), lambda i: (0, 0)),
            pl.BlockSpec((1, D), lambda i: (0, 0)),
        ],
        out_specs=pl.BlockSpec((nb, D), lambda i: (i, 0)),
        out_shape=jax.ShapeDtypeStruct((N_NODES, D), jnp.float32),
    )(nodes, agg, wn1, wn2, bn)

    return new_nodes, new_edges


def kernel(nodes, edges, senders, receivers, W_edge, b_edge, W_node, b_node):
    return _run(nodes, edges, senders, receivers,
                W_edge, b_edge, W_node, b_node)
